# baseline (speedup 1.0000x reference)
import functools

import jax
import jax.numpy as jnp
from jax import lax
from jax.experimental import pallas as pl
from jax.experimental.pallas import tpu as pltpu
from jax.experimental.pallas import tpu_sc as plsc

NC, NS = 2, 16
NW = NC * NS
CH = 16384
NBUF = 3


def _sc_copy_body(k_hbm, v_hbm, ko_hbm, vo_hbm, buf,
                  rs0, rs1, rs2, ws0, ws1, ws2):
    wid = lax.axis_index("s") * NC + lax.axis_index("c")
    n = k_hbm.shape[0]
    per_w = n // NW
    base = wid * per_w
    nchunks = per_w // CH
    rsems = (rs0, rs1, rs2)
    wsems = (ws0, ws1, ws2)
    jobs = []
    for src, dst in ((k_hbm, ko_hbm), (v_hbm, vo_hbm)):
        for c in range(nchunks):
            jobs.append((src, dst, c * CH))
    reads = [
        pltpu.make_async_copy(
            src.at[pl.ds(base + off, CH)], buf.at[i % NBUF], rsems[i % NBUF])
        for i, (src, dst, off) in enumerate(jobs)
    ]
    writes = [None] * len(jobs)
    for i in range(min(NBUF, len(jobs))):
        reads[i].start()
    for i, (src, dst, off) in enumerate(jobs):
        reads[i].wait()
        w = pltpu.make_async_copy(
            buf.at[i % NBUF], dst.at[pl.ds(base + off, CH)], wsems[i % NBUF])
        w.start()
        writes[i] = w
        if i + NBUF < len(jobs):
            writes[i].wait()
            reads[i + NBUF].start()
    for i in range(max(0, len(jobs) - NBUF), len(jobs)):
        writes[i].wait()


def sc_copy(k_flat, v_flat):
    n = k_flat.shape[0]
    mesh = plsc.VectorSubcoreMesh(core_axis_name="c", subcore_axis_name="s")
    fn = functools.partial(
        pl.kernel,
        mesh=mesh,
        out_type=[
            jax.ShapeDtypeStruct((n,), jnp.float32),
            jax.ShapeDtypeStruct((n,), jnp.float32),
        ],
        scratch_types=[
            pltpu.VMEM((NBUF, CH), jnp.float32),
            pltpu.SemaphoreType.DMA,
            pltpu.SemaphoreType.DMA,
            pltpu.SemaphoreType.DMA,
            pltpu.SemaphoreType.DMA,
            pltpu.SemaphoreType.DMA,
            pltpu.SemaphoreType.DMA,
        ],
    )(_sc_copy_body)
    return fn(k_flat, v_flat)


def kernel(k_val, v_val, k_cache, v_cache):
    del k_cache, v_cache
    b, s, h, d = k_val.shape
    n = b * s * h * d
    k_out, v_out = sc_copy(k_val.reshape(n), v_val.reshape(n))
    return (k_out.reshape(b, s, h, d), v_out.reshape(b, s, h, d))


# speedup vs baseline: 1.1292x; 1.1292x over previous
import functools

import jax
import jax.numpy as jnp
from jax import lax
from jax.experimental import pallas as pl
from jax.experimental.pallas import tpu as pltpu
from jax.experimental.pallas import tpu_sc as plsc

NC, NS = 2, 16
NW = NC * NS
CH = 32768
NBUF = 3


def _sc_copy_body(k_hbm, v_hbm, ko_hbm, vo_hbm, buf,
                  rs0, rs1, rs2, ws0, ws1, ws2):
    wid = lax.axis_index("s") * NC + lax.axis_index("c")
    n = k_hbm.shape[0]
    per_w = n // NW
    base = wid * per_w
    nchunks = per_w // CH
    rsems = (rs0, rs1, rs2)
    wsems = (ws0, ws1, ws2)
    jobs = []
    for src, dst in ((k_hbm, ko_hbm), (v_hbm, vo_hbm)):
        for c in range(nchunks):
            jobs.append((src, dst, c * CH))
    reads = [
        pltpu.make_async_copy(
            src.at[pl.ds(base + off, CH)], buf.at[i % NBUF], rsems[i % NBUF])
        for i, (src, dst, off) in enumerate(jobs)
    ]
    writes = [None] * len(jobs)
    for i in range(min(NBUF, len(jobs))):
        reads[i].start()
    for i, (src, dst, off) in enumerate(jobs):
        reads[i].wait()
        w = pltpu.make_async_copy(
            buf.at[i % NBUF], dst.at[pl.ds(base + off, CH)], wsems[i % NBUF])
        w.start()
        writes[i] = w
        if i + NBUF < len(jobs):
            writes[i].wait()
            reads[i + NBUF].start()
    for i in range(max(0, len(jobs) - NBUF), len(jobs)):
        writes[i].wait()


def sc_copy(k_flat, v_flat):
    n = k_flat.shape[0]
    mesh = plsc.VectorSubcoreMesh(core_axis_name="c", subcore_axis_name="s")
    fn = functools.partial(
        pl.kernel,
        mesh=mesh,
        out_type=[
            jax.ShapeDtypeStruct((n,), jnp.float32),
            jax.ShapeDtypeStruct((n,), jnp.float32),
        ],
        scratch_types=[
            pltpu.VMEM((NBUF, CH), jnp.float32),
            pltpu.SemaphoreType.DMA,
            pltpu.SemaphoreType.DMA,
            pltpu.SemaphoreType.DMA,
            pltpu.SemaphoreType.DMA,
            pltpu.SemaphoreType.DMA,
            pltpu.SemaphoreType.DMA,
        ],
    )(_sc_copy_body)
    return fn(k_flat, v_flat)


def kernel(k_val, v_val, k_cache, v_cache):
    del k_cache, v_cache
    b, s, h, d = k_val.shape
    n = b * s * h * d
    k_out, v_out = sc_copy(k_val.reshape(n), v_val.reshape(n))
    return (k_out.reshape(b, s, h, d), v_out.reshape(b, s, h, d))
